# baseline (speedup 1.0000x reference)
import jax
from jax import lax
import jax.numpy as jnp
from jax.experimental import pallas as pl
from jax.experimental.pallas import tpu as pltpu
from jax.experimental.pallas import tpu_sc as plsc

_N_NODES = 100000
_DIM = 128
_NUM_SPECIES = 128

_RB = 11776
_TC_BLOCKS = 8
_TC_ROWS = _RB * _TC_BLOCKS
_SC_ROWS = _N_NODES - _TC_ROWS
_MB = 4096
_SC_PAD = 2 * _MB
_SC_WINDOW = 128


def _sc_gather_tail(embeddings, idx2d):
    mesh = plsc.VectorSubcoreMesh(
        core_axis_name="core", subcore_axis_name="subcore"
    )

    @pl.kernel(
        out_type=jax.ShapeDtypeStruct((_SC_PAD, _DIM), embeddings.dtype),
        mesh=mesh,
    )
    def gather_kernel(x_hbm, i_hbm, o_hbm):
        def body(i_vmem, o_vmem):
            pltpu.sync_copy(x_hbm.at[i_vmem.at[0]], o_vmem)

        pltpu.emit_pipeline(
            body,
            grid=(_SC_PAD // _SC_WINDOW,),
            in_specs=[
                pl.BlockSpec((1, _SC_WINDOW), index_map=lambda i: (0, i))
            ],
            out_specs=[
                pl.BlockSpec((_SC_WINDOW, _DIM), index_map=lambda i: (i, 0))
            ],
            core_axis_name=("core", "subcore"),
            dimension_semantics=(pltpu.PARALLEL,),
        )(i_hbm, o_hbm)

    return gather_kernel(embeddings, idx2d)


def _tc_lookup_head(idx_head, embeddings):
    idxp = idx_head.reshape(_TC_BLOCKS, 1, _RB)

    def body(i_ref, w_ref, o_ref):
        ids = i_ref[0, 0, :]
        onehot = (
            ids[:, None]
            == lax.broadcasted_iota(jnp.int32, (_RB, _NUM_SPECIES), 1)
        ).astype(jnp.bfloat16)
        w = w_ref[...]
        w_hi = w.astype(jnp.bfloat16)
        r1 = w - w_hi.astype(jnp.float32)
        w_mid = r1.astype(jnp.bfloat16)
        acc = jnp.dot(onehot, w_hi, preferred_element_type=jnp.float32)
        acc = acc + jnp.dot(onehot, w_mid, preferred_element_type=jnp.float32)
        o_ref[...] = acc

    return pl.pallas_call(
        body,
        grid=(_TC_BLOCKS,),
        in_specs=[
            pl.BlockSpec((1, 1, _RB), lambda i: (i, 0, 0)),
            pl.BlockSpec((_NUM_SPECIES, _DIM), lambda i: (0, 0)),
        ],
        out_specs=pl.BlockSpec((_RB, _DIM), lambda i: (i, 0)),
        out_shape=jax.ShapeDtypeStruct((_N_NODES, _DIM), jnp.float32),
    )(idxp, embeddings)


def _merge_tail(tc_full, sc_part):
    def body(dst_any_ref, sc_ref, o_ref):
        del dst_any_ref
        o_ref[...] = sc_ref[...]

    return pl.pallas_call(
        body,
        grid=(_SC_PAD // _MB,),
        in_specs=[
            pl.BlockSpec(memory_space=pltpu.MemorySpace.HBM),
            pl.BlockSpec((_MB, _DIM), lambda i: (i, 0)),
        ],
        out_specs=pl.BlockSpec((_MB, _DIM), lambda i: (_TC_ROWS // _MB + i, 0)),
        out_shape=jax.ShapeDtypeStruct((_N_NODES, _DIM), jnp.float32),
        input_output_aliases={0: 0},
    )(tc_full, sc_part)


def kernel(node_specie, embeddings):
    idx_tail = jnp.pad(node_specie[_TC_ROWS:], (0, _SC_PAD - _SC_ROWS))
    sc_part = _sc_gather_tail(embeddings, idx_tail.reshape(1, _SC_PAD))
    tc_full = _tc_lookup_head(node_specie[:_TC_ROWS], embeddings)
    return _merge_tail(tc_full, sc_part)


# speedup vs baseline: 3.2533x; 3.2533x over previous
import jax
from jax import lax
import jax.numpy as jnp
from jax.experimental import pallas as pl
from jax.experimental.pallas import tpu as pltpu
from jax.experimental.pallas import tpu_sc as plsc

_N_NODES = 100000
_DIM = 128
_NUM_SPECIES = 128
_WINDOW = 128
_PADDED = 100096


def _sc_gather(embeddings, idx2d):
    mesh = plsc.VectorSubcoreMesh(
        core_axis_name="core", subcore_axis_name="subcore"
    )

    @pl.kernel(
        out_type=jax.ShapeDtypeStruct((_N_NODES, _DIM), embeddings.dtype),
        mesh=mesh,
        scratch_types=[
            pltpu.VMEM_SHARED((_NUM_SPECIES, _DIM), jnp.float32),
            pltpu.SemaphoreType.DMA,
        ],
    )
    def gather_kernel(x_hbm, i_hbm, o_hbm, tbl_vmem, sem):
        @pl.when(lax.axis_index("subcore") == 0)
        def _():
            pltpu.async_copy(x_hbm, tbl_vmem, sem).wait()

        plsc.subcore_barrier()

        def body(i_vmem, o_vmem):
            pltpu.sync_copy(tbl_vmem.at[i_vmem.at[0]], o_vmem)

        pltpu.emit_pipeline(
            body,
            grid=(_PADDED // _WINDOW,),
            in_specs=[pl.BlockSpec((1, _WINDOW), index_map=lambda i: (0, i))],
            out_specs=[
                pl.BlockSpec((_WINDOW, _DIM), index_map=lambda i: (i, 0))
            ],
            core_axis_name=("core", "subcore"),
            dimension_semantics=(pltpu.PARALLEL,),
        )(i_hbm, o_hbm)

    return gather_kernel(embeddings, idx2d)


def kernel(node_specie, embeddings):
    idx = jnp.pad(node_specie, (0, _PADDED - _N_NODES))
    return _sc_gather(embeddings, idx.reshape(1, _PADDED))


# speedup vs baseline: 3.2602x; 1.0021x over previous
import jax
from jax import lax
import jax.numpy as jnp
from jax.experimental import pallas as pl
from jax.experimental.pallas import tpu as pltpu
from jax.experimental.pallas import tpu_sc as plsc

_N_NODES = 100000
_DIM = 128
_NUM_SPECIES = 128
_WINDOW = 256
_PADDED = 100096


def _sc_gather(embeddings, idx2d):
    mesh = plsc.VectorSubcoreMesh(
        core_axis_name="core", subcore_axis_name="subcore"
    )

    @pl.kernel(
        out_type=jax.ShapeDtypeStruct((_N_NODES, _DIM), embeddings.dtype),
        mesh=mesh,
        scratch_types=[
            pltpu.VMEM_SHARED((_NUM_SPECIES, _DIM), jnp.float32),
            pltpu.SemaphoreType.DMA,
        ],
    )
    def gather_kernel(x_hbm, i_hbm, o_hbm, tbl_vmem, sem):
        @pl.when(lax.axis_index("subcore") == 0)
        def _():
            pltpu.async_copy(x_hbm, tbl_vmem, sem).wait()

        plsc.subcore_barrier()

        def body(i_vmem, o_vmem):
            pltpu.sync_copy(tbl_vmem.at[i_vmem.at[0]], o_vmem)

        pltpu.emit_pipeline(
            body,
            grid=(_PADDED // _WINDOW,),
            in_specs=[pl.BlockSpec((1, _WINDOW), index_map=lambda i: (0, i))],
            out_specs=[
                pl.BlockSpec((_WINDOW, _DIM), index_map=lambda i: (i, 0))
            ],
            core_axis_name=("core", "subcore"),
            dimension_semantics=(pltpu.PARALLEL,),
        )(i_hbm, o_hbm)

    return gather_kernel(embeddings, idx2d)


def kernel(node_specie, embeddings):
    idx = jnp.pad(node_specie, (0, _PADDED - _N_NODES))
    return _sc_gather(embeddings, idx.reshape(1, _PADDED))
